# baseline (speedup 1.0000x reference)
import jax
import jax.numpy as jnp
from jax import lax
from jax.experimental import pallas as pl
from jax.experimental.pallas import tpu as pltpu
from jax.experimental.pallas import tpu_sc as plsc

B = 4096
D = 128
NODES = 50000
DNODE = 1024
E = 500000

NTILES = 32
EPT = 32768
EPAD = 16 * EPT
EB = 2048
MAPN = 50016
SLOTS = 2 * B
HALF = SLOTS // 2
ACC_ROWS = 4224
STRIPE = ACC_ROWS // 16
CH = 64
NBUF = 4
RCAP = 4096
SPT = SLOTS // NTILES


def _drug_body(x_ref, w_ref, b_ref, o_ref):
    o_ref[...] = lax.dot_general(
        x_ref[...], w_ref[...], (((1,), (0,)), ((), ())),
        preferred_element_type=jnp.float32) + b_ref[...]


def _drug_encoder(drug_input, W_drug, b_drug):
    return pl.pallas_call(
        _drug_body,
        out_shape=jax.ShapeDtypeStruct((B, D), jnp.float32),
    )(drug_input, W_drug, b_drug.reshape(1, D))


def _main_body(fd_ref, ep_ref, zd_ref, zp_ref,
               wg_ref, bg_ref,
               fl1_ref, flb1_ref, fl2_ref, flb2_ref, fl3_ref, flb3_ref,
               fr1_ref, frb1_ref, fr2_ref, frb2_ref, fr3_ref, frb3_ref,
               ow_ref, ob_ref, out_ref):
    bf16 = jnp.bfloat16
    dot = lambda a, b: lax.dot_general(
        a, b, (((1,), (0,)), ((), ())), preferred_element_type=jnp.float32)
    bdot = lambda a, b: lax.dot_general(
        a.astype(bf16), b, (((1,), (0,)), ((), ())),
        preferred_element_type=jnp.float32)
    yd = jnp.maximum(bdot(zd_ref[...], wg_ref[...]) + bg_ref[...], 0.0)
    yp = jnp.maximum(bdot(zp_ref[...], wg_ref[...]) + bg_ref[...], 0.0)
    h2 = jnp.maximum(bdot(yd, fr1_ref[0:DNODE]) + bdot(yp, fr1_ref[DNODE:2 * DNODE])
                     + frb1_ref[...], 0.0)
    h2 = jnp.maximum(bdot(h2, fr2_ref[...]) + frb2_ref[...], 0.0)
    o2 = dot(h2, fr3_ref[...]) + frb3_ref[...]
    h1 = jnp.maximum(bdot(fd_ref[...], fl1_ref[0:D]) + bdot(ep_ref[...], fl1_ref[D:2 * D])
                     + flb1_ref[...], 0.0)
    h1 = jnp.maximum(bdot(h1, fl2_ref[...]) + flb2_ref[...], 0.0)
    o1 = dot(h1, fl3_ref[...]) + flb3_ref[...]
    out_ref[...] = o1 * ow_ref[0, 0] + o2 * ow_ref[1, 0] + ob_ref[0, 0]


def _dense_main(fd, ep, z, W_gnn, b_gnn,
                fcl_w1, fcl_b1, fcl_w2, fcl_b2, fcl_w3, fcl_b3,
                fcr_w1, fcr_b1, fcr_w2, fcr_b2, fcr_w3, fcr_b3,
                out_w, out_b):
    BLK = 512
    grid = (B // BLK,)
    row = pl.BlockSpec((BLK, D), lambda i: (i, 0))
    zd_spec = pl.BlockSpec((BLK, D), lambda i: (i, 0))
    zp_spec = pl.BlockSpec((BLK, D), lambda i: (i + B // BLK, 0))
    full = lambda shape: pl.BlockSpec(shape, lambda i: (0,) * len(shape))
    return pl.pallas_call(
        _main_body,
        grid=grid,
        in_specs=[
            row, row, zd_spec, zp_spec,
            full((D, DNODE)), full((1, DNODE)),
            full((2 * D, 1024)), full((1, 1024)),
            full((1024, 512)), full((1, 512)),
            full((512, 1)), full((1, 1)),
            full((2 * DNODE, 1024)), full((1, 1024)),
            full((1024, 512)), full((1, 512)),
            full((512, 1)), full((1, 1)),
            full((2, 1)), full((1, 1)),
        ],
        out_specs=pl.BlockSpec((BLK, 1), lambda i: (i, 0)),
        out_shape=jax.ShapeDtypeStruct((B, 1), jnp.float32),
    )(fd, ep, z, z, W_gnn.astype(jnp.bfloat16), b_gnn.reshape(1, DNODE),
      fcl_w1.astype(jnp.bfloat16), fcl_b1.reshape(1, 1024),
      fcl_w2.astype(jnp.bfloat16), fcl_b2.reshape(1, 512),
      fcl_w3, fcl_b3.reshape(1, 1),
      fcr_w1.astype(jnp.bfloat16), fcr_b1.reshape(1, 1024),
      fcr_w2.astype(jnp.bfloat16), fcr_b2.reshape(1, 512),
      fcr_w3, fcr_b3.reshape(1, 1),
      out_w, out_b.reshape(1, 1))


_MESH = dict(core_axis_name="c", subcore_axis_name="s", num_cores=2,
             num_subcores=16)


def _edge_body(enc_hbm, dst_hbm, src_hbm, ids_hbm, tx_hbm, z0_hbm,
               acc_hbm, rep_hbm,
               mapv, dstb, srcb, ring, rowbuf, tstage, sstage,
               idsv, repbuf, accum, sem):
    c = lax.axis_index("c")
    s = lax.axis_index("s")
    wid = c * 16 + s
    i32 = jnp.int32
    iota = lax.iota(i32, 16)

    pltpu.sync_copy(z0_hbm, accum.at[pl.ds(s * STRIPE, STRIPE)])
    pltpu.sync_copy(enc_hbm, mapv)
    plsc.subcore_barrier()

    def append16(off, entry, mm):
        cm = plsc.cumsum(jnp.where(mm, jnp.int32(1), jnp.int32(0)))
        pos = jnp.maximum(off + cm - 1, 0) & (RCAP - 1)
        plsc.store_scatter(ring, [pos], entry, mask=mm)
        return off + jnp.max(cm)

    def retire(n):
        b = n & (NBUF - 1)
        pltpu.make_async_copy(tx_hbm.at[pl.ds(0, CH)],
                              rowbuf.at[b], sem).wait()
        pltpu.sync_copy(rowbuf.at[b], accum.at[sstage.at[b]], add=True)

    def drain_chunks(wptr, dptr):
        nch = lax.shift_right_logical(wptr - dptr, 6)
        n0 = lax.shift_right_logical(dptr, 6)

        def db(k, _):
            n = n0 + k
            b = n & (NBUF - 1)

            @pl.when(n >= NBUF - 1)
            def _():
                retire(n - (NBUF - 1))

            base = (dptr + k * CH) & (RCAP - 1)

            def ub(j, _2):
                e = ring[pl.ds(base + j * 16, 16)]
                encs = plsc.load_gather(mapv, [e & 0xFFFF])
                tstage[b, pl.ds(j * 16, 16)] = encs & 0xFFFF
                sstage[b, pl.ds(j * 16, 16)] = lax.shift_right_logical(e, 16)
                return 0

            lax.fori_loop(0, CH // 16, ub, 0, unroll=4)
            pltpu.async_copy(tx_hbm.at[tstage.at[b]], rowbuf.at[b], sem)
            return 0

        lax.fori_loop(0, nch, db, 0)
        return dptr + nch * CH

    pltpu.sync_copy(ids_hbm.at[pl.ds(wid * SPT, SPT)], idsv)

    def slot_body(j, off):
        iv = idsv[pl.ds(j * 16, 16)]
        enc = plsc.load_gather(mapv, [iv])
        r = lax.shift_right_logical(enc, 16) - 1
        radj = r + lax.shift_right_logical(r, 12) * (ACC_ROWS - HALF)
        repbuf[pl.ds(j * 16, 16)] = radj
        kvec = wid * SPT + j * 16 + iota
        entry = ((kvec & (HALF - 1)) << 16) | iv
        return append16(off, entry, r == kvec)

    off = lax.fori_loop(0, SPT // 16, slot_body, jnp.int32(0))
    pltpu.sync_copy(repbuf, rep_hbm.at[pl.ds(wid * SPT, SPT)])

    def blk_body(b, carry):
        off, dr = carry
        base = s * EPT + b * EB
        pltpu.sync_copy(dst_hbm.at[pl.ds(base, EB)], dstb)
        pltpu.sync_copy(src_hbm.at[pl.ds(base, EB)], srcb)

        def vb(i, off2):
            d = dstb[pl.ds(i * 16, 16)]
            sv = srcb[pl.ds(i * 16, 16)]
            encd = plsc.load_gather(mapv, [d])
            rep1 = lax.shift_right_logical(encd, 16)
            slot = rep1 - 1
            m = (rep1 > 0) & (lax.shift_right_logical(slot, 12) == c)
            entry = ((slot & (HALF - 1)) << 16) | sv
            return append16(off2, entry, m)

        off = lax.fori_loop(0, EB // 16, vb, off, unroll=4)
        return (off, drain_chunks(off, dr))

    off, dr = lax.fori_loop(0, EPT // EB, blk_body,
                            (off, jnp.int32(0)))

    dummy = jnp.broadcast_to(jnp.int32(HALF << 16), (16,))
    nr = (off + (CH - 1)) & ~(CH - 1)

    def fb(j, _):
        pos = off + j * 16 + iota
        plsc.store_scatter(ring, [pos & (RCAP - 1)], dummy, mask=pos < nr)
        return 0

    lax.fori_loop(0, CH // 16, fb, 0)
    drain_chunks(nr, dr)

    ntot = lax.shift_right_logical(nr, 6)
    lax.fori_loop(jnp.maximum(ntot - (NBUF - 1), 0), ntot,
                  lambda t, _: (retire(t), 0)[1], 0)

    plsc.subcore_barrier()

    pltpu.sync_copy(accum.at[pl.ds(s * STRIPE, STRIPE)],
                    acc_hbm.at[pl.ds(c * ACC_ROWS + s * STRIPE, STRIPE)])


def _sc_edge(enc, dst_p, src_p, ids, tx, zrows):
    f32 = jnp.float32
    i32 = jnp.int32
    fn = pl.kernel(
        _edge_body,
        out_type=[
            jax.ShapeDtypeStruct((2 * ACC_ROWS, D), f32),
            jax.ShapeDtypeStruct((SLOTS,), i32),
        ],
        mesh=plsc.VectorSubcoreMesh(**_MESH),
        compiler_params=pltpu.CompilerParams(needs_layout_passes=False),
        scratch_types=[
            pltpu.VMEM((MAPN,), i32),
            pltpu.VMEM((EB,), i32),
            pltpu.VMEM((EB,), i32),
            pltpu.VMEM((RCAP,), i32),
            pltpu.VMEM((NBUF, CH, D), f32),
            pltpu.VMEM((NBUF, CH), i32),
            pltpu.VMEM((NBUF, CH), i32),
            pltpu.VMEM((SPT,), i32),
            pltpu.VMEM((SPT,), i32),
            pltpu.VMEM_SHARED((ACC_ROWS, D), f32),
            pltpu.SemaphoreType.DMA,
        ],
    )
    return fn(enc, dst_p, src_p, ids, tx, zrows)


def _comb_body(acc_hbm, rep_hbm, z_hbm, repv, buf, sem):
    c = lax.axis_index("c")
    s = lax.axis_index("s")
    wid = c * 16 + s
    pltpu.sync_copy(rep_hbm.at[pl.ds(wid * 4, 4)], repv)
    for j in range(4):
        pltpu.async_copy(acc_hbm.at[repv.at[j]],
                         buf.at[pl.ds(j * CH, CH)], sem)
    pltpu.make_async_copy(acc_hbm.at[pl.ds(0, SPT)], buf, sem).wait()
    pltpu.sync_copy(buf, z_hbm.at[pl.ds(wid * SPT, SPT)])


def _sc_combine(acc, rep2):
    fn = pl.kernel(
        _comb_body,
        out_type=jax.ShapeDtypeStruct((SLOTS, D), jnp.float32),
        mesh=plsc.VectorSubcoreMesh(**_MESH),
        compiler_params=pltpu.CompilerParams(needs_layout_passes=False),
        scratch_types=[
            pltpu.VMEM((4, CH), jnp.int32),
            pltpu.VMEM((SPT, D), jnp.float32),
            pltpu.SemaphoreType.DMA,
        ],
    )
    return fn(acc, rep2)


def kernel(drug_input, protein_ids, pair_index, edge_index,
           W_drug, b_drug, protein_table, node_feature,
           fcl_w1, fcl_b1, fcl_w2, fcl_b2, fcl_w3, fcl_b3,
           W_gnn, b_gnn,
           fcr_w1, fcr_b1, fcr_w2, fcr_b2, fcr_w3, fcr_b3,
           out_w, out_b):
    i32 = jnp.int32
    drug_id = pair_index[:, 0].astype(i32)
    protein_id = pair_index[:, 1].astype(i32)
    src = edge_index[0].astype(i32)
    dst = edge_index[1].astype(i32)
    pids = protein_ids.astype(i32)

    fd = _drug_encoder(drug_input, W_drug, b_drug)
    ep = jnp.take(protein_table, pids, axis=0)
    tx = jnp.concatenate([node_feature, fd, ep], axis=0)

    ii = jnp.arange(B, dtype=i32)
    wmap = jnp.full((NODES,), -1, i32)
    wmap = wmap.at[drug_id[::-1]].set(ii[::-1])
    wmap = wmap.at[protein_id[::-1]].set((ii + B)[::-1])
    ids = jnp.concatenate([drug_id, protein_id])
    rep = jnp.full((NODES,), -1, i32).at[ids].set(jnp.arange(SLOTS, dtype=i32))
    nid = jnp.arange(NODES, dtype=i32)
    tidx = jnp.where(wmap >= 0, NODES + wmap, nid)
    enc = ((rep + 1) << 16) | tidx
    enc = jnp.concatenate([enc, jnp.zeros((MAPN - NODES,), i32)])

    dst_p = jnp.concatenate([dst, jnp.full((EPAD - E,), NODES, i32)])
    src_p = jnp.concatenate([src, jnp.zeros((EPAD - E,), i32)])
    zrows = jnp.zeros((STRIPE, D), jnp.float32)

    acc, rep_out = _sc_edge(enc, dst_p, src_p, ids, tx, zrows)
    z = _sc_combine(acc, rep_out.reshape(SLOTS // 64, 64))

    return _dense_main(fd, ep, z, W_gnn, b_gnn,
                       fcl_w1, fcl_b1, fcl_w2, fcl_b2, fcl_w3, fcl_b3,
                       fcr_w1, fcr_b1, fcr_w2, fcr_b2, fcr_w3, fcr_b3,
                       out_w, out_b)


# speedup vs baseline: 1.1422x; 1.1422x over previous
import jax
import jax.numpy as jnp
from jax import lax
from jax.experimental import pallas as pl
from jax.experimental.pallas import tpu as pltpu
from jax.experimental.pallas import tpu_sc as plsc

B = 4096
D = 128
NODES = 50000
DNODE = 1024
E = 500000

NTILES = 32
EPT = 32768
EPAD = 16 * EPT
EB = 2048
MAPN = 50016
SLOTS = 2 * B
HALF = SLOTS // 2
ACC_ROWS = 4224
STRIPE = ACC_ROWS // 16
CH = 64
NBUF = 4
RCAP = 4096
SPT = SLOTS // NTILES


def _drug_body(x_ref, w_ref, b_ref, o_ref):
    o_ref[...] = lax.dot_general(
        x_ref[...], w_ref[...], (((1,), (0,)), ((), ())),
        preferred_element_type=jnp.float32) + b_ref[...]


def _drug_encoder(drug_input, W_drug, b_drug):
    return pl.pallas_call(
        _drug_body,
        out_shape=jax.ShapeDtypeStruct((B, D), jnp.float32),
    )(drug_input, W_drug, b_drug.reshape(1, D))


def _main_body(fd_ref, ep_ref, zd_ref, zp_ref,
               wg_ref, bg_ref,
               fl1_ref, flb1_ref, fl2_ref, flb2_ref, fl3_ref, flb3_ref,
               fr1_ref, frb1_ref, fr2_ref, frb2_ref, fr3_ref, frb3_ref,
               ow_ref, ob_ref, out_ref):
    dot = lambda a, b: lax.dot_general(
        a, b, (((1,), (0,)), ((), ())), preferred_element_type=jnp.float32)
    yd = jnp.maximum(dot(zd_ref[...], wg_ref[...]) + bg_ref[...], 0.0)
    yp = jnp.maximum(dot(zp_ref[...], wg_ref[...]) + bg_ref[...], 0.0)
    h2 = jnp.maximum(dot(yd, fr1_ref[0:DNODE]) + dot(yp, fr1_ref[DNODE:2 * DNODE])
                     + frb1_ref[...], 0.0)
    h2 = jnp.maximum(dot(h2, fr2_ref[...]) + frb2_ref[...], 0.0)
    o2 = dot(h2, fr3_ref[...]) + frb3_ref[...]
    h1 = jnp.maximum(dot(fd_ref[...], fl1_ref[0:D]) + dot(ep_ref[...], fl1_ref[D:2 * D])
                     + flb1_ref[...], 0.0)
    h1 = jnp.maximum(dot(h1, fl2_ref[...]) + flb2_ref[...], 0.0)
    o1 = dot(h1, fl3_ref[...]) + flb3_ref[...]
    out_ref[...] = o1 * ow_ref[0, 0] + o2 * ow_ref[1, 0] + ob_ref[0, 0]


def _dense_main(fd, ep, z, W_gnn, b_gnn,
                fcl_w1, fcl_b1, fcl_w2, fcl_b2, fcl_w3, fcl_b3,
                fcr_w1, fcr_b1, fcr_w2, fcr_b2, fcr_w3, fcr_b3,
                out_w, out_b):
    BLK = 512
    grid = (B // BLK,)
    row = pl.BlockSpec((BLK, D), lambda i: (i, 0))
    zd_spec = pl.BlockSpec((BLK, D), lambda i: (i, 0))
    zp_spec = pl.BlockSpec((BLK, D), lambda i: (i + B // BLK, 0))
    full = lambda shape: pl.BlockSpec(shape, lambda i: (0,) * len(shape))
    return pl.pallas_call(
        _main_body,
        grid=grid,
        in_specs=[
            row, row, zd_spec, zp_spec,
            full((D, DNODE)), full((1, DNODE)),
            full((2 * D, 1024)), full((1, 1024)),
            full((1024, 512)), full((1, 512)),
            full((512, 1)), full((1, 1)),
            full((2 * DNODE, 1024)), full((1, 1024)),
            full((1024, 512)), full((1, 512)),
            full((512, 1)), full((1, 1)),
            full((2, 1)), full((1, 1)),
        ],
        out_specs=pl.BlockSpec((BLK, 1), lambda i: (i, 0)),
        out_shape=jax.ShapeDtypeStruct((B, 1), jnp.float32),
    )(fd, ep, z, z, W_gnn, b_gnn.reshape(1, DNODE),
      fcl_w1, fcl_b1.reshape(1, 1024), fcl_w2, fcl_b2.reshape(1, 512),
      fcl_w3, fcl_b3.reshape(1, 1), fcr_w1, fcr_b1.reshape(1, 1024),
      fcr_w2, fcr_b2.reshape(1, 512), fcr_w3, fcr_b3.reshape(1, 1),
      out_w, out_b.reshape(1, 1))


_MESH = dict(core_axis_name="c", subcore_axis_name="s", num_cores=2,
             num_subcores=16)


def _edge_body(enc_hbm, dst_hbm, src_hbm, ids_hbm, tx_hbm, z0_hbm,
               acc_hbm,
               mapv, dstb, srcb, ring, rowbuf, tstage, sstage,
               idsv, accum, sem):
    c = lax.axis_index("c")
    s = lax.axis_index("s")
    wid = c * 16 + s
    i32 = jnp.int32
    iota = lax.iota(i32, 16)

    pltpu.sync_copy(z0_hbm, accum.at[pl.ds(s * STRIPE, STRIPE)])
    pltpu.sync_copy(enc_hbm.at[c], mapv)
    plsc.subcore_barrier()

    def append16(off, entry, mm):
        cm = plsc.cumsum(jnp.where(mm, jnp.int32(1), jnp.int32(0)))
        pos = (off + cm - 1) & (RCAP - 1)
        plsc.store_scatter(ring, [pos], entry, mask=mm)
        return off + plsc.all_reduce_population_count(mm)

    def retire(n):
        b = n & (NBUF - 1)
        pltpu.make_async_copy(tx_hbm.at[pl.ds(0, CH)],
                              rowbuf.at[b], sem).wait()
        pltpu.sync_copy(rowbuf.at[b], accum.at[sstage.at[b]], add=True)

    def drain_chunks(wptr, dptr):
        nch = lax.shift_right_logical(wptr - dptr, 6)
        n0 = lax.shift_right_logical(dptr, 6)

        def db(k, _):
            n = n0 + k
            b = n & (NBUF - 1)

            @pl.when(n >= NBUF - 1)
            def _():
                retire(n - (NBUF - 1))

            base = (dptr + k * CH) & (RCAP - 1)

            def ub(j, _2):
                e = ring[pl.ds(base + j * 16, 16)]
                encs = plsc.load_gather(mapv, [e & 0xFFFF])
                tstage[b, pl.ds(j * 16, 16)] = encs & 0xFFFF
                sstage[b, pl.ds(j * 16, 16)] = lax.shift_right_logical(e, 16)
                return 0

            lax.fori_loop(0, CH // 16, ub, 0, unroll=4)
            pltpu.async_copy(tx_hbm.at[tstage.at[b]], rowbuf.at[b], sem)
            return 0

        lax.fori_loop(0, nch, db, 0)
        return dptr + nch * CH

    pltpu.sync_copy(ids_hbm.at[pl.ds(wid * SPT, SPT)], idsv)
    zero16 = jnp.broadcast_to(jnp.int32(0), (16,))

    def slot_body(j, off):
        iv = idsv[pl.ds(j * 16, 16)]
        enc = plsc.load_gather(mapv, [iv])
        rep1 = lax.shift_right_logical(enc, 16)
        lk = (wid & 15) * SPT + j * 16 + iota
        entry = (lk << 16) | iv
        return append16(off, entry, rep1 == lk + 1)

    off = lax.fori_loop(0, SPT // 16, slot_body, zero16)

    def blk_body(b, carry):
        off, dr = carry
        base = s * EPT + b * EB
        pltpu.sync_copy(dst_hbm.at[pl.ds(base, EB)], dstb)
        pltpu.sync_copy(src_hbm.at[pl.ds(base, EB)], srcb)

        def vb(i, off2):
            d = dstb[pl.ds(i * 16, 16)]
            sv = srcb[pl.ds(i * 16, 16)]
            encd = plsc.load_gather(mapv, [d])
            rep1 = lax.shift_right_logical(encd, 16)
            entry = ((rep1 - 1) << 16) | sv
            return append16(off2, entry, rep1 > 0)

        off = lax.fori_loop(0, EB // 16, vb, off, unroll=4)
        return (off, drain_chunks(jnp.max(off), dr))

    off, dr = lax.fori_loop(0, EPT // EB, blk_body,
                            (off, jnp.int32(0)))
    offs = jnp.max(off)

    dummy = jnp.broadcast_to(jnp.int32(HALF << 16), (16,))
    nr = (offs + (CH - 1)) & ~(CH - 1)

    def fb(j, _):
        pos = offs + j * 16 + iota
        plsc.store_scatter(ring, [pos & (RCAP - 1)], dummy, mask=pos < nr)
        return 0

    lax.fori_loop(0, CH // 16, fb, 0)
    drain_chunks(nr, dr)

    ntot = lax.shift_right_logical(nr, 6)
    lax.fori_loop(jnp.maximum(ntot - (NBUF - 1), 0), ntot,
                  lambda t, _: (retire(t), 0)[1], 0)

    plsc.subcore_barrier()

    pltpu.sync_copy(accum.at[pl.ds(s * STRIPE, STRIPE)],
                    acc_hbm.at[pl.ds(c * ACC_ROWS + s * STRIPE, STRIPE)])


def _sc_edge(enc2, dst_p, src_p, ids, tx, zrows):
    f32 = jnp.float32
    i32 = jnp.int32
    fn = pl.kernel(
        _edge_body,
        out_type=jax.ShapeDtypeStruct((2 * ACC_ROWS, D), f32),
        mesh=plsc.VectorSubcoreMesh(**_MESH),
        compiler_params=pltpu.CompilerParams(needs_layout_passes=False),
        scratch_types=[
            pltpu.VMEM((MAPN,), i32),
            pltpu.VMEM((EB,), i32),
            pltpu.VMEM((EB,), i32),
            pltpu.VMEM((RCAP,), i32),
            pltpu.VMEM((NBUF, CH, D), f32),
            pltpu.VMEM((NBUF, CH), i32),
            pltpu.VMEM((NBUF, CH), i32),
            pltpu.VMEM((SPT,), i32),
            pltpu.VMEM_SHARED((ACC_ROWS, D), f32),
            pltpu.SemaphoreType.DMA,
        ],
    )
    return fn(enc2, dst_p, src_p, ids, tx, zrows)


def _comb_body(base_hbm, ids_hbm, acc_hbm, z_hbm, mapv, idsv, repv, buf, sem):
    c = lax.axis_index("c")
    s = lax.axis_index("s")
    wid = c * 16 + s
    i32 = jnp.int32
    pltpu.sync_copy(base_hbm, mapv)
    pltpu.sync_copy(ids_hbm.at[pl.ds(wid * SPT, SPT)], idsv)

    def tb(j, _):
        iv = idsv[pl.ds(j * 16, 16)]
        enc = plsc.load_gather(mapv, [iv])
        r = lax.shift_right_logical(enc, 16) - 1
        radj = r + lax.shift_right_logical(r, 12) * (ACC_ROWS - HALF)
        repv[j >> 2, pl.ds((j & 3) * 16, 16)] = radj
        return 0

    lax.fori_loop(0, SPT // 16, tb, 0, unroll=4)
    for j in range(SPT // CH):
        pltpu.async_copy(acc_hbm.at[repv.at[j]],
                         buf.at[pl.ds(j * CH, CH)], sem)
    pltpu.make_async_copy(acc_hbm.at[pl.ds(0, SPT)], buf, sem).wait()
    pltpu.sync_copy(buf, z_hbm.at[pl.ds(wid * SPT, SPT)])


def _sc_combine(base, ids, acc):
    fn = pl.kernel(
        _comb_body,
        out_type=jax.ShapeDtypeStruct((SLOTS, D), jnp.float32),
        mesh=plsc.VectorSubcoreMesh(**_MESH),
        compiler_params=pltpu.CompilerParams(needs_layout_passes=False),
        scratch_types=[
            pltpu.VMEM((MAPN,), jnp.int32),
            pltpu.VMEM((SPT,), jnp.int32),
            pltpu.VMEM((SPT // CH, CH), jnp.int32),
            pltpu.VMEM((SPT, D), jnp.float32),
            pltpu.SemaphoreType.DMA,
        ],
    )
    return fn(base, ids, acc)


def kernel(drug_input, protein_ids, pair_index, edge_index,
           W_drug, b_drug, protein_table, node_feature,
           fcl_w1, fcl_b1, fcl_w2, fcl_b2, fcl_w3, fcl_b3,
           W_gnn, b_gnn,
           fcr_w1, fcr_b1, fcr_w2, fcr_b2, fcr_w3, fcr_b3,
           out_w, out_b):
    i32 = jnp.int32
    drug_id = pair_index[:, 0].astype(i32)
    protein_id = pair_index[:, 1].astype(i32)
    src = edge_index[0].astype(i32)
    dst = edge_index[1].astype(i32)
    pids = protein_ids.astype(i32)

    fd = _drug_encoder(drug_input, W_drug, b_drug)
    ep = jnp.take(protein_table, pids, axis=0)
    tx = jnp.concatenate([node_feature, fd, ep], axis=0)

    ii = jnp.arange(B, dtype=i32)
    ids = jnp.concatenate([drug_id, protein_id])
    upd_idx = jnp.concatenate([drug_id[::-1], protein_id[::-1]])
    upd_val = jnp.concatenate([
        (((ii + 1) << 16) | (NODES + ii))[::-1],
        (((B + ii + 1) << 16) | (NODES + B + ii))[::-1],
    ])
    base = jnp.arange(MAPN, dtype=i32).at[upd_idx].set(upd_val)
    rep1 = lax.shift_right_logical(base, 16)
    tidx = base & 0xFFFF
    lsl1 = ((rep1 - 1) & (HALF - 1)) + 1
    in0 = (rep1 > 0) & (rep1 <= HALF)
    in1 = rep1 > HALF
    enc2 = jnp.stack([
        jnp.where(in0, (lsl1 << 16) | tidx, tidx),
        jnp.where(in1, (lsl1 << 16) | tidx, tidx),
    ])

    dst_p = jnp.concatenate([dst, jnp.full((EPAD - E,), NODES, i32)])
    src_p = jnp.concatenate([src, jnp.zeros((EPAD - E,), i32)])
    zrows = jnp.zeros((STRIPE, D), jnp.float32)

    acc = _sc_edge(enc2, dst_p, src_p, ids, tx, zrows)
    z = _sc_combine(base, ids, acc)

    return _dense_main(fd, ep, z, W_gnn, b_gnn,
                       fcl_w1, fcl_b1, fcl_w2, fcl_b2, fcl_w3, fcl_b3,
                       fcr_w1, fcr_b1, fcr_w2, fcr_b2, fcr_w3, fcr_b3,
                       out_w, out_b)


# speedup vs baseline: 1.2261x; 1.0735x over previous
import jax
import jax.numpy as jnp
from jax import lax
from jax.experimental import pallas as pl
from jax.experimental.pallas import tpu as pltpu
from jax.experimental.pallas import tpu_sc as plsc

B = 4096
D = 128
NODES = 50000
DNODE = 1024
E = 500000

NTILES = 32
EPT = 32768
EPAD = 16 * EPT
EB = 2048
MAPN = 50016
SLOTS = 2 * B
HALF = SLOTS // 2
ACC_ROWS = 4224
STRIPE = ACC_ROWS // 16
CH = 64
NBUF = 4
RCAP = 4096
SPT = SLOTS // NTILES


def _drug_body(x_ref, w_ref, b_ref, o_ref):
    o_ref[...] = lax.dot_general(
        x_ref[...], w_ref[...], (((1,), (0,)), ((), ())),
        preferred_element_type=jnp.float32) + b_ref[...]


def _drug_encoder(drug_input, W_drug, b_drug):
    return pl.pallas_call(
        _drug_body,
        out_shape=jax.ShapeDtypeStruct((B, D), jnp.float32),
    )(drug_input, W_drug, b_drug.reshape(1, D))


def _left_body(fd_ref, ep_ref,
               fl1_ref, flb1_ref, fl2_ref, flb2_ref, fl3_ref, flb3_ref,
               o1_ref):
    dot = lambda a, b: lax.dot_general(
        a, b, (((1,), (0,)), ((), ())), preferred_element_type=jnp.float32)
    h1 = jnp.maximum(dot(fd_ref[...], fl1_ref[0:D]) + dot(ep_ref[...], fl1_ref[D:2 * D])
                     + flb1_ref[...], 0.0)
    h1 = jnp.maximum(dot(h1, fl2_ref[...]) + flb2_ref[...], 0.0)
    o1_ref[...] = dot(h1, fl3_ref[...]) + flb3_ref[...]


def _dense_left(fd, ep, fcl_w1, fcl_b1, fcl_w2, fcl_b2, fcl_w3, fcl_b3):
    BLK = 512
    row = pl.BlockSpec((BLK, D), lambda i: (i, 0))
    full = lambda shape: pl.BlockSpec(shape, lambda i: (0,) * len(shape))
    return pl.pallas_call(
        _left_body,
        grid=(B // BLK,),
        in_specs=[
            row, row,
            full((2 * D, 1024)), full((1, 1024)),
            full((1024, 512)), full((1, 512)),
            full((512, 1)), full((1, 1)),
        ],
        out_specs=pl.BlockSpec((BLK, 1), lambda i: (i, 0)),
        out_shape=jax.ShapeDtypeStruct((B, 1), jnp.float32),
    )(fd, ep, fcl_w1, fcl_b1.reshape(1, 1024), fcl_w2, fcl_b2.reshape(1, 512),
      fcl_w3, fcl_b3.reshape(1, 1))


def _main_body(zd_ref, zp_ref, o1_ref,
               wg_ref, bg_ref,
               fr1_ref, frb1_ref, fr2_ref, frb2_ref, fr3_ref, frb3_ref,
               ow_ref, ob_ref, out_ref):
    dot = lambda a, b: lax.dot_general(
        a, b, (((1,), (0,)), ((), ())), preferred_element_type=jnp.float32)
    yd = jnp.maximum(dot(zd_ref[...], wg_ref[...]) + bg_ref[...], 0.0)
    yp = jnp.maximum(dot(zp_ref[...], wg_ref[...]) + bg_ref[...], 0.0)
    h2 = jnp.maximum(dot(yd, fr1_ref[0:DNODE]) + dot(yp, fr1_ref[DNODE:2 * DNODE])
                     + frb1_ref[...], 0.0)
    h2 = jnp.maximum(dot(h2, fr2_ref[...]) + frb2_ref[...], 0.0)
    o2 = dot(h2, fr3_ref[...]) + frb3_ref[...]
    out_ref[...] = (o1_ref[...] * ow_ref[0, 0] + o2 * ow_ref[1, 0]
                    + ob_ref[0, 0])


def _dense_main(z, o1, W_gnn, b_gnn,
                fcr_w1, fcr_b1, fcr_w2, fcr_b2, fcr_w3, fcr_b3,
                out_w, out_b):
    BLK = 512
    grid = (B // BLK,)
    zd_spec = pl.BlockSpec((BLK, D), lambda i: (i, 0))
    zp_spec = pl.BlockSpec((BLK, D), lambda i: (i + B // BLK, 0))
    full = lambda shape: pl.BlockSpec(shape, lambda i: (0,) * len(shape))
    return pl.pallas_call(
        _main_body,
        grid=grid,
        in_specs=[
            zd_spec, zp_spec, pl.BlockSpec((BLK, 1), lambda i: (i, 0)),
            full((D, DNODE)), full((1, DNODE)),
            full((2 * DNODE, 1024)), full((1, 1024)),
            full((1024, 512)), full((1, 512)),
            full((512, 1)), full((1, 1)),
            full((2, 1)), full((1, 1)),
        ],
        out_specs=pl.BlockSpec((BLK, 1), lambda i: (i, 0)),
        out_shape=jax.ShapeDtypeStruct((B, 1), jnp.float32),
    )(z, z, o1, W_gnn, b_gnn.reshape(1, DNODE),
      fcr_w1, fcr_b1.reshape(1, 1024), fcr_w2, fcr_b2.reshape(1, 512),
      fcr_w3, fcr_b3.reshape(1, 1), out_w, out_b.reshape(1, 1))


_MESH = dict(core_axis_name="c", subcore_axis_name="s", num_cores=2,
             num_subcores=16)


def _edge_body(enc_hbm, dst_hbm, src_hbm, ids_hbm, tx_hbm, z0_hbm,
               acc_hbm,
               mapv, dstb, srcb, ring, rowbuf, tstage, sstage,
               idsv, accum, sem, sem2):
    c = lax.axis_index("c")
    s = lax.axis_index("s")
    wid = c * 16 + s
    i32 = jnp.int32
    iota = lax.iota(i32, 16)

    pltpu.sync_copy(z0_hbm, accum.at[pl.ds(s * STRIPE, STRIPE)])
    pltpu.sync_copy(enc_hbm.at[c], mapv)
    plsc.subcore_barrier()

    def append16(off, entry, mm):
        cm = plsc.cumsum(jnp.where(mm, jnp.int32(1), jnp.int32(0)))
        pos = (off + cm - 1) & (RCAP - 1)
        plsc.store_scatter(ring, [pos], entry, mask=mm)
        return off + plsc.all_reduce_population_count(mm)

    def retire(n):
        b = n & (NBUF - 1)
        pltpu.make_async_copy(tx_hbm.at[pl.ds(0, CH)],
                              rowbuf.at[b], sem).wait()
        pltpu.sync_copy(rowbuf.at[b], accum.at[sstage.at[b]], add=True)

    def drain_chunks(wptr, dptr):
        nch = lax.shift_right_logical(wptr - dptr, 6)
        n0 = lax.shift_right_logical(dptr, 6)

        def db(k, _):
            n = n0 + k
            b = n & (NBUF - 1)

            @pl.when(n >= NBUF - 1)
            def _():
                retire(n - (NBUF - 1))

            base = (dptr + k * CH) & (RCAP - 1)

            def ub(j, _2):
                e = ring[pl.ds(base + j * 16, 16)]
                encs = plsc.load_gather(mapv, [e & 0xFFFF])
                tstage[b, pl.ds(j * 16, 16)] = encs & 0xFFFF
                sstage[b, pl.ds(j * 16, 16)] = lax.shift_right_logical(e, 16)
                return 0

            lax.fori_loop(0, CH // 16, ub, 0, unroll=4)
            pltpu.async_copy(tx_hbm.at[tstage.at[b]], rowbuf.at[b], sem)
            return 0

        lax.fori_loop(0, nch, db, 0)
        return dptr + nch * CH

    pltpu.sync_copy(ids_hbm.at[pl.ds(wid * SPT, SPT)], idsv)
    zero16 = jnp.broadcast_to(jnp.int32(0), (16,))

    def slot_body(j, off):
        iv = idsv[pl.ds(j * 16, 16)]
        enc = plsc.load_gather(mapv, [iv])
        rep1 = lax.shift_right_logical(enc, 16)
        lk = (wid & 15) * SPT + j * 16 + iota
        entry = (lk << 16) | iv
        return append16(off, entry, rep1 == lk + 1)

    off = lax.fori_loop(0, SPT // 16, slot_body, zero16)

    NBLK = EPT // EB

    def stage(b):
        p = b & 1
        pltpu.async_copy(dst_hbm.at[pl.ds(s * EPT + b * EB, EB)],
                         dstb.at[p], sem2)
        pltpu.async_copy(src_hbm.at[pl.ds(s * EPT + b * EB, EB)],
                         srcb.at[p], sem2)

    stage(jnp.int32(0))

    def blk_body(b, carry):
        off, dr = carry
        p = b & 1
        pltpu.make_async_copy(dst_hbm.at[pl.ds(0, EB)], dstb.at[p],
                              sem2).wait()
        pltpu.make_async_copy(src_hbm.at[pl.ds(0, EB)], srcb.at[p],
                              sem2).wait()

        @pl.when(b + 1 < NBLK)
        def _():
            stage(b + 1)

        def vb(i, off2):
            d = dstb[p, pl.ds(i * 16, 16)]
            sv = srcb[p, pl.ds(i * 16, 16)]
            encd = plsc.load_gather(mapv, [d])
            rep1 = lax.shift_right_logical(encd, 16)
            entry = ((rep1 - 1) << 16) | sv
            return append16(off2, entry, rep1 > 0)

        off = lax.fori_loop(0, EB // 16, vb, off, unroll=4)
        return (off, drain_chunks(jnp.max(off), dr))

    off, dr = lax.fori_loop(0, NBLK, blk_body,
                            (off, jnp.int32(0)))
    offs = jnp.max(off)

    dummy = jnp.broadcast_to(jnp.int32(HALF << 16), (16,))
    nr = (offs + (CH - 1)) & ~(CH - 1)

    def fb(j, _):
        pos = offs + j * 16 + iota
        plsc.store_scatter(ring, [pos & (RCAP - 1)], dummy, mask=pos < nr)
        return 0

    lax.fori_loop(0, CH // 16, fb, 0)
    drain_chunks(nr, dr)

    ntot = lax.shift_right_logical(nr, 6)
    lax.fori_loop(jnp.maximum(ntot - (NBUF - 1), 0), ntot,
                  lambda t, _: (retire(t), 0)[1], 0)

    plsc.subcore_barrier()

    pltpu.sync_copy(accum.at[pl.ds(s * STRIPE, STRIPE)],
                    acc_hbm.at[pl.ds(c * ACC_ROWS + s * STRIPE, STRIPE)])


def _sc_edge(enc2, dst_p, src_p, ids, tx, zrows):
    f32 = jnp.float32
    i32 = jnp.int32
    fn = pl.kernel(
        _edge_body,
        out_type=jax.ShapeDtypeStruct((2 * ACC_ROWS, D), f32),
        mesh=plsc.VectorSubcoreMesh(**_MESH),
        compiler_params=pltpu.CompilerParams(needs_layout_passes=False),
        scratch_types=[
            pltpu.VMEM((MAPN,), i32),
            pltpu.VMEM((2, EB), i32),
            pltpu.VMEM((2, EB), i32),
            pltpu.VMEM((RCAP,), i32),
            pltpu.VMEM((NBUF, CH, D), f32),
            pltpu.VMEM((NBUF, CH), i32),
            pltpu.VMEM((NBUF, CH), i32),
            pltpu.VMEM((SPT,), i32),
            pltpu.VMEM_SHARED((ACC_ROWS, D), f32),
            pltpu.SemaphoreType.DMA,
            pltpu.SemaphoreType.DMA,
        ],
    )
    return fn(enc2, dst_p, src_p, ids, tx, zrows)


def _comb_body(base_hbm, ids_hbm, acc_hbm, z_hbm, mapv, idsv, repv, buf, sem):
    c = lax.axis_index("c")
    s = lax.axis_index("s")
    wid = c * 16 + s
    i32 = jnp.int32
    pltpu.sync_copy(base_hbm, mapv)
    pltpu.sync_copy(ids_hbm.at[pl.ds(wid * SPT, SPT)], idsv)

    def tb(j, _):
        iv = idsv[pl.ds(j * 16, 16)]
        enc = plsc.load_gather(mapv, [iv])
        r = lax.shift_right_logical(enc, 16) - 1
        radj = r + lax.shift_right_logical(r, 12) * (ACC_ROWS - HALF)
        repv[j >> 2, pl.ds((j & 3) * 16, 16)] = radj
        return 0

    lax.fori_loop(0, SPT // 16, tb, 0, unroll=4)
    for j in range(SPT // CH):
        pltpu.async_copy(acc_hbm.at[repv.at[j]],
                         buf.at[pl.ds(j * CH, CH)], sem)
    pltpu.make_async_copy(acc_hbm.at[pl.ds(0, SPT)], buf, sem).wait()
    pltpu.sync_copy(buf, z_hbm.at[pl.ds(wid * SPT, SPT)])


def _sc_combine(base, ids, acc):
    fn = pl.kernel(
        _comb_body,
        out_type=jax.ShapeDtypeStruct((SLOTS, D), jnp.float32),
        mesh=plsc.VectorSubcoreMesh(**_MESH),
        compiler_params=pltpu.CompilerParams(needs_layout_passes=False),
        scratch_types=[
            pltpu.VMEM((MAPN,), jnp.int32),
            pltpu.VMEM((SPT,), jnp.int32),
            pltpu.VMEM((SPT // CH, CH), jnp.int32),
            pltpu.VMEM((SPT, D), jnp.float32),
            pltpu.SemaphoreType.DMA,
        ],
    )
    return fn(base, ids, acc)


def kernel(drug_input, protein_ids, pair_index, edge_index,
           W_drug, b_drug, protein_table, node_feature,
           fcl_w1, fcl_b1, fcl_w2, fcl_b2, fcl_w3, fcl_b3,
           W_gnn, b_gnn,
           fcr_w1, fcr_b1, fcr_w2, fcr_b2, fcr_w3, fcr_b3,
           out_w, out_b):
    i32 = jnp.int32
    drug_id = pair_index[:, 0].astype(i32)
    protein_id = pair_index[:, 1].astype(i32)
    src = edge_index[0].astype(i32)
    dst = edge_index[1].astype(i32)
    pids = protein_ids.astype(i32)

    fd = _drug_encoder(drug_input, W_drug, b_drug)
    ep = jnp.take(protein_table, pids, axis=0)
    tx = jnp.concatenate([node_feature, fd, ep], axis=0)

    ii = jnp.arange(B, dtype=i32)
    ids = jnp.concatenate([drug_id, protein_id])
    upd_idx = jnp.concatenate([drug_id[::-1], protein_id[::-1]])
    upd_val = jnp.concatenate([
        (((ii + 1) << 16) | (NODES + ii))[::-1],
        (((B + ii + 1) << 16) | (NODES + B + ii))[::-1],
    ])
    base = jnp.arange(MAPN, dtype=i32).at[upd_idx].set(upd_val)
    rep1 = lax.shift_right_logical(base, 16)
    tidx = base & 0xFFFF
    lsl1 = ((rep1 - 1) & (HALF - 1)) + 1
    in0 = (rep1 > 0) & (rep1 <= HALF)
    in1 = rep1 > HALF
    enc2 = jnp.stack([
        jnp.where(in0, (lsl1 << 16) | tidx, tidx),
        jnp.where(in1, (lsl1 << 16) | tidx, tidx),
    ])

    dst_p = jnp.concatenate([dst, jnp.full((EPAD - E,), NODES, i32)])
    src_p = jnp.concatenate([src, jnp.zeros((EPAD - E,), i32)])
    zrows = jnp.zeros((STRIPE, D), jnp.float32)

    o1 = _dense_left(fd, ep, fcl_w1, fcl_b1, fcl_w2, fcl_b2, fcl_w3, fcl_b3)
    acc = _sc_edge(enc2, dst_p, src_p, ids, tx, zrows)
    z = _sc_combine(base, ids, acc)

    return _dense_main(z, o1, W_gnn, b_gnn,
                       fcr_w1, fcr_b1, fcr_w2, fcr_b2, fcr_w3, fcr_b3,
                       out_w, out_b)


# speedup vs baseline: 1.2263x; 1.0001x over previous
import jax
import jax.numpy as jnp
from jax import lax
from jax.experimental import pallas as pl
from jax.experimental.pallas import tpu as pltpu
from jax.experimental.pallas import tpu_sc as plsc

B = 4096
D = 128
NODES = 50000
DNODE = 1024
E = 500000

NTILES = 32
EPT = 32768
EPAD = 16 * EPT
EB = 2048
MAPN = 50016
SLOTS = 2 * B
HALF = SLOTS // 2
ACC_ROWS = 4224
STRIPE = ACC_ROWS // 16
CH = 64
NBUF = 4
RCAP = 4096
SPT = SLOTS // NTILES


def _drug_body(x_ref, w_ref, b_ref, o_ref):
    o_ref[...] = lax.dot_general(
        x_ref[...], w_ref[...], (((1,), (0,)), ((), ())),
        preferred_element_type=jnp.float32) + b_ref[...]


def _drug_encoder(drug_input, W_drug, b_drug):
    return pl.pallas_call(
        _drug_body,
        out_shape=jax.ShapeDtypeStruct((B, D), jnp.float32),
    )(drug_input, W_drug, b_drug.reshape(1, D))


def _left_body(fd_ref, ep_ref,
               fl1_ref, flb1_ref, fl2_ref, flb2_ref, fl3_ref, flb3_ref,
               o1_ref):
    dot = lambda a, b: lax.dot_general(
        a, b, (((1,), (0,)), ((), ())), preferred_element_type=jnp.float32)
    h1 = jnp.maximum(dot(fd_ref[...], fl1_ref[0:D]) + dot(ep_ref[...], fl1_ref[D:2 * D])
                     + flb1_ref[...], 0.0)
    h1 = jnp.maximum(dot(h1, fl2_ref[...]) + flb2_ref[...], 0.0)
    o1_ref[...] = dot(h1, fl3_ref[...]) + flb3_ref[...]


def _dense_left(fd, ep, fcl_w1, fcl_b1, fcl_w2, fcl_b2, fcl_w3, fcl_b3):
    BLK = 512
    row = pl.BlockSpec((BLK, D), lambda i: (i, 0))
    full = lambda shape: pl.BlockSpec(shape, lambda i: (0,) * len(shape))
    return pl.pallas_call(
        _left_body,
        grid=(B // BLK,),
        in_specs=[
            row, row,
            full((2 * D, 1024)), full((1, 1024)),
            full((1024, 512)), full((1, 512)),
            full((512, 1)), full((1, 1)),
        ],
        out_specs=pl.BlockSpec((BLK, 1), lambda i: (i, 0)),
        out_shape=jax.ShapeDtypeStruct((B, 1), jnp.float32),
    )(fd, ep, fcl_w1, fcl_b1.reshape(1, 1024), fcl_w2, fcl_b2.reshape(1, 512),
      fcl_w3, fcl_b3.reshape(1, 1))


def _main_body(zd_ref, zp_ref, o1_ref,
               wg_ref, bg_ref,
               fr1_ref, frb1_ref, fr2_ref, frb2_ref, fr3_ref, frb3_ref,
               ow_ref, ob_ref, out_ref):
    dot = lambda a, b: lax.dot_general(
        a, b, (((1,), (0,)), ((), ())), preferred_element_type=jnp.float32)
    yd = jnp.maximum(dot(zd_ref[...], wg_ref[...]) + bg_ref[...], 0.0)
    yp = jnp.maximum(dot(zp_ref[...], wg_ref[...]) + bg_ref[...], 0.0)
    h2 = jnp.maximum(dot(yd, fr1_ref[0:DNODE]) + dot(yp, fr1_ref[DNODE:2 * DNODE])
                     + frb1_ref[...], 0.0)
    h2 = jnp.maximum(dot(h2, fr2_ref[...]) + frb2_ref[...], 0.0)
    o2 = dot(h2, fr3_ref[...]) + frb3_ref[...]
    out_ref[...] = (o1_ref[...] * ow_ref[0, 0] + o2 * ow_ref[1, 0]
                    + ob_ref[0, 0])


def _dense_main(z, o1, W_gnn, b_gnn,
                fcr_w1, fcr_b1, fcr_w2, fcr_b2, fcr_w3, fcr_b3,
                out_w, out_b):
    BLK = 512
    grid = (B // BLK,)
    zd_spec = pl.BlockSpec((BLK, D), lambda i: (i, 0))
    zp_spec = pl.BlockSpec((BLK, D), lambda i: (i + B // BLK, 0))
    full = lambda shape: pl.BlockSpec(shape, lambda i: (0,) * len(shape))
    return pl.pallas_call(
        _main_body,
        grid=grid,
        in_specs=[
            zd_spec, zp_spec, pl.BlockSpec((BLK, 1), lambda i: (i, 0)),
            full((D, DNODE)), full((1, DNODE)),
            full((2 * DNODE, 1024)), full((1, 1024)),
            full((1024, 512)), full((1, 512)),
            full((512, 1)), full((1, 1)),
            full((2, 1)), full((1, 1)),
        ],
        out_specs=pl.BlockSpec((BLK, 1), lambda i: (i, 0)),
        out_shape=jax.ShapeDtypeStruct((B, 1), jnp.float32),
    )(z, z, o1, W_gnn, b_gnn.reshape(1, DNODE),
      fcr_w1, fcr_b1.reshape(1, 1024), fcr_w2, fcr_b2.reshape(1, 512),
      fcr_w3, fcr_b3.reshape(1, 1), out_w, out_b.reshape(1, 1))


_MESH = dict(core_axis_name="c", subcore_axis_name="s", num_cores=2,
             num_subcores=16)


def _edge_body(enc_hbm, dst_hbm, src_hbm, ids_hbm, tx_hbm, z0_hbm,
               acc_hbm,
               mapv, dstb, srcb, ring, rowbuf, tstage, sstage,
               idsv, accum, sem, sem2):
    c = lax.axis_index("c")
    s = lax.axis_index("s")
    wid = c * 16 + s
    i32 = jnp.int32
    iota = lax.iota(i32, 16)

    pltpu.sync_copy(z0_hbm, accum.at[pl.ds(s * STRIPE, STRIPE)])
    pltpu.sync_copy(enc_hbm.at[c], mapv)
    plsc.subcore_barrier()

    def append16(off, entry, mm):
        cm = plsc.cumsum(jnp.where(mm, jnp.int32(1), jnp.int32(0)))
        pos = (off + cm - 1) & (RCAP - 1)
        plsc.store_scatter(ring, [pos], entry, mask=mm)
        return off + plsc.all_reduce_population_count(mm)

    def retire(n):
        b = n & (NBUF - 1)
        pltpu.make_async_copy(tx_hbm.at[pl.ds(0, CH)],
                              rowbuf.at[b], sem).wait()
        pltpu.sync_copy(rowbuf.at[b], accum.at[sstage.at[b]], add=True)

    def drain_chunks(wptr, dptr):
        nch = lax.shift_right_logical(wptr - dptr, 6)
        n0 = lax.shift_right_logical(dptr, 6)

        def db(k, _):
            n = n0 + k
            b = n & (NBUF - 1)

            @pl.when(n >= NBUF - 1)
            def _():
                retire(n - (NBUF - 1))

            base = (dptr + k * CH) & (RCAP - 1)

            def ub(j, _2):
                e = ring[pl.ds(base + j * 16, 16)]
                encs = plsc.load_gather(mapv, [e & 0xFFFF])
                tstage[b, pl.ds(j * 16, 16)] = encs & 0xFFFF
                sstage[b, pl.ds(j * 16, 16)] = lax.shift_right_logical(e, 16)
                return 0

            lax.fori_loop(0, CH // 16, ub, 0, unroll=4)
            pltpu.async_copy(tx_hbm.at[tstage.at[b]], rowbuf.at[b], sem)
            return 0

        lax.fori_loop(0, nch, db, 0)
        return dptr + nch * CH

    pltpu.sync_copy(ids_hbm.at[pl.ds(wid * SPT, SPT)], idsv)
    zero16 = jnp.broadcast_to(jnp.int32(0), (16,))

    def slot_body(j, off):
        iv = idsv[pl.ds(j * 16, 16)]
        enc = plsc.load_gather(mapv, [iv])
        rep1 = lax.shift_right_logical(enc, 16)
        lk = (wid & 15) * SPT + j * 16 + iota
        entry = (lk << 16) | iv
        return append16(off, entry, rep1 == lk + 1)

    off = lax.fori_loop(0, SPT // 16, slot_body, zero16)

    NBLK = EPT // EB

    def stage(b):
        p = b & 1
        pltpu.async_copy(dst_hbm.at[pl.ds(s * EPT + b * EB, EB)],
                         dstb.at[p], sem2)
        pltpu.async_copy(src_hbm.at[pl.ds(s * EPT + b * EB, EB)],
                         srcb.at[p], sem2)

    stage(jnp.int32(0))

    def blk_body(b, carry):
        off, dr = carry
        p = b & 1
        pltpu.make_async_copy(dst_hbm.at[pl.ds(0, EB)], dstb.at[p],
                              sem2).wait()
        pltpu.make_async_copy(src_hbm.at[pl.ds(0, EB)], srcb.at[p],
                              sem2).wait()

        @pl.when(b + 1 < NBLK)
        def _():
            stage(b + 1)

        def vb(i, off2):
            d = dstb[p, pl.ds(i * 16, 16)]
            sv = srcb[p, pl.ds(i * 16, 16)]
            encd = plsc.load_gather(mapv, [d])
            rep1 = lax.shift_right_logical(encd, 16)
            entry = ((rep1 - 1) << 16) | sv
            return append16(off2, entry, rep1 > 0)

        off = lax.fori_loop(0, EB // 16, vb, off, unroll=4)
        return (off, drain_chunks(jnp.max(off), dr))

    off, dr = lax.fori_loop(0, NBLK, blk_body,
                            (off, jnp.int32(0)))
    offs = jnp.max(off)

    dummy = jnp.broadcast_to(jnp.int32(HALF << 16), (16,))
    nr = (offs + (CH - 1)) & ~(CH - 1)

    def fb(j, _):
        pos = offs + j * 16 + iota
        plsc.store_scatter(ring, [pos & (RCAP - 1)], dummy, mask=pos < nr)
        return 0

    lax.fori_loop(0, CH // 16, fb, 0)
    drain_chunks(nr, dr)

    ntot = lax.shift_right_logical(nr, 6)
    lax.fori_loop(jnp.maximum(ntot - (NBUF - 1), 0), ntot,
                  lambda t, _: (retire(t), 0)[1], 0)

    plsc.subcore_barrier()

    pltpu.sync_copy(accum.at[pl.ds(s * STRIPE, STRIPE)],
                    acc_hbm.at[pl.ds(c * ACC_ROWS + s * STRIPE, STRIPE)])


def _sc_edge(enc2, dst_p, src_p, ids, tx, zrows):
    f32 = jnp.float32
    i32 = jnp.int32
    fn = pl.kernel(
        _edge_body,
        out_type=jax.ShapeDtypeStruct((2 * ACC_ROWS, D), f32),
        mesh=plsc.VectorSubcoreMesh(**_MESH),
        compiler_params=pltpu.CompilerParams(needs_layout_passes=False),
        scratch_types=[
            pltpu.VMEM((MAPN,), i32),
            pltpu.VMEM((2, EB), i32),
            pltpu.VMEM((2, EB), i32),
            pltpu.VMEM((RCAP,), i32),
            pltpu.VMEM((NBUF, CH, D), f32),
            pltpu.VMEM((NBUF, CH), i32),
            pltpu.VMEM((NBUF, CH), i32),
            pltpu.VMEM((SPT,), i32),
            pltpu.VMEM_SHARED((ACC_ROWS, D), f32),
            pltpu.SemaphoreType.DMA,
            pltpu.SemaphoreType.DMA,
        ],
    )
    return fn(enc2, dst_p, src_p, ids, tx, zrows)


def _comb_body(base_hbm, ids_hbm, acc_hbm, z_hbm, mapv, idsv, repv, buf, sem):
    c = lax.axis_index("c")
    s = lax.axis_index("s")
    wid = c * 16 + s
    i32 = jnp.int32
    pltpu.sync_copy(base_hbm, mapv)
    pltpu.sync_copy(ids_hbm.at[pl.ds(wid * SPT, SPT)], idsv)

    def tb(j, _):
        iv = idsv[pl.ds(j * 16, 16)]
        enc = plsc.load_gather(mapv, [iv])
        r = lax.shift_right_logical(enc, 16) - 1
        radj = r + lax.shift_right_logical(r, 12) * (ACC_ROWS - HALF)
        repv[j >> 2, pl.ds((j & 3) * 16, 16)] = radj
        return 0

    lax.fori_loop(0, SPT // 16, tb, 0, unroll=4)
    for j in range(SPT // CH):
        pltpu.async_copy(acc_hbm.at[repv.at[j]],
                         buf.at[pl.ds(j * CH, CH)], sem)
    pltpu.make_async_copy(acc_hbm.at[pl.ds(0, SPT)], buf, sem).wait()
    pltpu.sync_copy(buf, z_hbm.at[pl.ds(wid * SPT, SPT)])


def _sc_combine(base, ids, acc):
    fn = pl.kernel(
        _comb_body,
        out_type=jax.ShapeDtypeStruct((SLOTS, D), jnp.float32),
        mesh=plsc.VectorSubcoreMesh(**_MESH),
        compiler_params=pltpu.CompilerParams(needs_layout_passes=False),
        scratch_types=[
            pltpu.VMEM((MAPN,), jnp.int32),
            pltpu.VMEM((SPT,), jnp.int32),
            pltpu.VMEM((SPT // CH, CH), jnp.int32),
            pltpu.VMEM((SPT, D), jnp.float32),
            pltpu.SemaphoreType.DMA,
        ],
    )
    return fn(base, ids, acc)


def kernel(drug_input, protein_ids, pair_index, edge_index,
           W_drug, b_drug, protein_table, node_feature,
           fcl_w1, fcl_b1, fcl_w2, fcl_b2, fcl_w3, fcl_b3,
           W_gnn, b_gnn,
           fcr_w1, fcr_b1, fcr_w2, fcr_b2, fcr_w3, fcr_b3,
           out_w, out_b):
    i32 = jnp.int32
    drug_id = pair_index[:, 0].astype(i32)
    protein_id = pair_index[:, 1].astype(i32)
    src = edge_index[0].astype(i32)
    dst = edge_index[1].astype(i32)
    pids = protein_ids.astype(i32)

    fd = _drug_encoder(drug_input, W_drug, b_drug)
    ep = jnp.take(protein_table, pids, axis=0)
    tx = jnp.concatenate([node_feature, fd, ep], axis=0)

    ii = jnp.arange(B, dtype=i32)
    ids = jnp.concatenate([drug_id, protein_id])
    upd_idx = jnp.concatenate([drug_id[::-1], protein_id[::-1]])
    upd_val = jnp.concatenate([
        (((ii + 1) << 16) | (NODES + ii))[::-1],
        (((B + ii + 1) << 16) | (NODES + B + ii))[::-1],
    ])
    base = jnp.arange(MAPN, dtype=i32).at[upd_idx].set(upd_val)
    rep1 = lax.shift_right_logical(base, 16)
    tidx = base & 0xFFFF
    lsl1 = ((rep1 - 1) & (HALF - 1)) + 1
    in0 = (rep1 > 0) & (rep1 <= HALF)
    in1 = rep1 > HALF
    enc2 = jnp.stack([
        jnp.where(in0, (lsl1 << 16) | tidx, tidx),
        jnp.where(in1, (lsl1 << 16) | tidx, tidx),
    ])

    dst_p = jnp.concatenate([dst, jnp.full((EPAD - E,), NODES, i32)])
    src_p = jnp.concatenate([src, jnp.zeros((EPAD - E,), i32)])
    zrows = jnp.zeros((STRIPE, D), jnp.float32)

    acc = _sc_edge(enc2, dst_p, src_p, ids, tx, zrows)
    o1 = _dense_left(fd, ep, fcl_w1, fcl_b1, fcl_w2, fcl_b2, fcl_w3, fcl_b3)
    z = _sc_combine(base, ids, acc)

    return _dense_main(z, o1, W_gnn, b_gnn,
                       fcr_w1, fcr_b1, fcr_w2, fcr_b2, fcr_w3, fcr_b3,
                       out_w, out_b)
